# baseline (speedup 1.0000x reference)
import functools

import jax
import jax.numpy as jnp
from jax import lax
from jax.experimental import pallas as pl
from jax.experimental.pallas import tpu as pltpu
from jax.experimental.pallas import tpu_sc as plsc

N = 10000
E = 320000
D = 128
DH = D // 2

NC = 2
NS = 16
CHUNK = 128
SUP = 4

_cpt = -(-E // (NS * CHUNK))
CPT = _cpt + (1 - _cpt % 2)
E_PAD = NS * CPT * CHUNK

RPT = -(-(N + 1) // (NS * 128)) * 128
NPAD = RPT * NS

_MESH = plsc.VectorSubcoreMesh(core_axis_name="c", subcore_axis_name="s")


@functools.partial(
    pl.kernel,
    out_type=jax.ShapeDtypeStruct((NC * NPAD,), jnp.float32),
    mesh=_MESH,
    scratch_types=[
        pltpu.VMEM((CPT, CHUNK), jnp.int32),
        pltpu.VMEM((CHUNK,), jnp.float32),
        pltpu.VMEM_SHARED((NPAD,), jnp.float32),
    ],
)
def _deg_kernel(col_hbm, zeros1_hbm, out_hbm, col_v, ones_v, acc):
    c = lax.axis_index("c")
    s = lax.axis_index("s")
    pltpu.sync_copy(col_hbm.at[s], col_v)
    for i in range(CHUNK // 16):
        ones_v[pl.ds(i * 16, 16)] = jnp.ones((16,), jnp.float32)
    pltpu.sync_copy(zeros1_hbm, acc.at[pl.ds(s * RPT, RPT)])
    plsc.subcore_barrier()

    def body(j, carry):
        pltpu.sync_copy(ones_v, acc.at[col_v.at[j]], add=True)
        return carry

    lax.fori_loop(0, CPT, body, 0)
    plsc.subcore_barrier()
    pltpu.sync_copy(acc.at[pl.ds(s * RPT, RPT)],
                    out_hbm.at[pl.ds(c * NPAD + s * RPT, RPT)])


@functools.partial(
    pl.kernel,
    out_type=jax.ShapeDtypeStruct((NC, NPAD, DH), jnp.float32),
    mesh=_MESH,
    scratch_types=[
        pltpu.VMEM((CPT, CHUNK), jnp.int32),
        pltpu.VMEM((CPT, CHUNK), jnp.int32),
        pltpu.VMEM((CHUNK, DH), jnp.float32),
        pltpu.VMEM((CHUNK, DH), jnp.float32),
        pltpu.VMEM_SHARED((NPAD, DH), jnp.float32),
        pltpu.SemaphoreType.DMA,
        pltpu.SemaphoreType.DMA,
    ],
    compiler_params=pltpu.CompilerParams(use_tc_tiling_on_sc=False),
)
def _msg_kernel(xw_hbm, row_hbm, col_hbm, zeros2_hbm, out_hbm,
                row_v, col_v, buf0, buf1, acc, sem0, sem1):
    c = lax.axis_index("c")
    s = lax.axis_index("s")
    xw_c = xw_hbm.at[c]
    pltpu.sync_copy(row_hbm.at[s], row_v)
    pltpu.sync_copy(col_hbm.at[s], col_v)
    pltpu.sync_copy(zeros2_hbm, acc.at[pl.ds(s * RPT, RPT)])
    plsc.subcore_barrier()

    pltpu.async_copy(xw_c.at[row_v.at[0]], buf0, sem0)

    def pair(i, carry):
        j = 2 * i
        pltpu.make_async_copy(xw_c.at[row_v.at[j]], buf0, sem0).wait()
        pltpu.async_copy(xw_c.at[row_v.at[j + 1]], buf1, sem1)
        pltpu.sync_copy(buf0, acc.at[col_v.at[j]], add=True)
        pltpu.make_async_copy(xw_c.at[row_v.at[j + 1]], buf1, sem1).wait()
        pltpu.async_copy(xw_c.at[row_v.at[j + 2]], buf0, sem0)
        pltpu.sync_copy(buf1, acc.at[col_v.at[j + 1]], add=True)
        return carry

    lax.fori_loop(0, (CPT - 1) // 2, pair, 0)
    pltpu.make_async_copy(xw_c.at[row_v.at[CPT - 1]], buf0, sem0).wait()
    pltpu.sync_copy(buf0, acc.at[col_v.at[CPT - 1]], add=True)

    plsc.subcore_barrier()
    pltpu.sync_copy(acc.at[pl.ds(s * RPT, RPT)],
                    out_hbm.at[c, pl.ds(s * RPT, RPT)])


BN = 2000


def _tc1a_body(x_ref, w1_ref, xw_ref):
    xw_ref[...] = jnp.dot(x_ref[...], w1_ref[...],
                          preferred_element_type=jnp.float32)


_tc1a = pl.pallas_call(
    _tc1a_body,
    grid=(N // BN,),
    in_specs=[
        pl.BlockSpec((BN, D), lambda i: (i, 0)),
        pl.BlockSpec((D, D), lambda i: (0, 0)),
    ],
    out_specs=pl.BlockSpec((BN, D), lambda i: (i, 0)),
    out_shape=jax.ShapeDtypeStruct((N, D), jnp.float32),
)


def _tc1b_body(deg_ref, xw_ref, xws_ref, dis_ref):
    dis = lax.rsqrt(deg_ref[...] + 1.0)
    v = dis * xw_ref[...]
    xws_ref[0] = v[:, :DH]
    xws_ref[1] = v[:, DH:]
    dis_ref[...] = dis


_tc1b = pl.pallas_call(
    _tc1b_body,
    grid=(N // BN,),
    in_specs=[
        pl.BlockSpec((BN, 1), lambda i: (i, 0)),
        pl.BlockSpec((BN, D), lambda i: (i, 0)),
    ],
    out_specs=[
        pl.BlockSpec((NC, BN, DH), lambda i: (0, i, 0)),
        pl.BlockSpec((BN, 1), lambda i: (i, 0)),
    ],
    out_shape=[
        jax.ShapeDtypeStruct((NC, N, DH), jnp.float32),
        jax.ShapeDtypeStruct((N, 1), jnp.float32),
    ],
)


def _tc2_body(a0_ref, a1_ref, x0_ref, x1_ref, dis_ref, b_ref, w_ref, out_ref):
    dis = dis_ref[...]
    pre = jnp.concatenate(
        [a0_ref[0] + x0_ref[0], a1_ref[0] + x1_ref[0]], axis=1)
    h = jnp.maximum(dis * pre + b_ref[...], 0.0)
    v = dis * jnp.dot(h, w_ref[...], preferred_element_type=jnp.float32)
    out_ref[0] = v[:, :DH]
    out_ref[1] = v[:, DH:]


_tc2 = pl.pallas_call(
    _tc2_body,
    grid=(N // BN,),
    in_specs=[
        pl.BlockSpec((1, BN, DH), lambda i: (0, i, 0)),
        pl.BlockSpec((1, BN, DH), lambda i: (1, i, 0)),
        pl.BlockSpec((1, BN, DH), lambda i: (0, i, 0)),
        pl.BlockSpec((1, BN, DH), lambda i: (1, i, 0)),
        pl.BlockSpec((BN, 1), lambda i: (i, 0)),
        pl.BlockSpec((1, D), lambda i: (0, 0)),
        pl.BlockSpec((D, D), lambda i: (0, 0)),
    ],
    out_specs=pl.BlockSpec((NC, BN, DH), lambda i: (0, i, 0)),
    out_shape=jax.ShapeDtypeStruct((NC, N, DH), jnp.float32),
)


def _tc3_body(a0_ref, a1_ref, x0_ref, x1_ref, dis_ref, b_ref, out_ref):
    dis = dis_ref[...]
    pre = jnp.concatenate(
        [a0_ref[0] + x0_ref[0], a1_ref[0] + x1_ref[0]], axis=1)
    out_ref[...] = jnp.maximum(dis * pre + b_ref[...], 0.0)


_tc3 = pl.pallas_call(
    _tc3_body,
    grid=(N // BN,),
    in_specs=[
        pl.BlockSpec((1, BN, DH), lambda i: (0, i, 0)),
        pl.BlockSpec((1, BN, DH), lambda i: (1, i, 0)),
        pl.BlockSpec((1, BN, DH), lambda i: (0, i, 0)),
        pl.BlockSpec((1, BN, DH), lambda i: (1, i, 0)),
        pl.BlockSpec((BN, 1), lambda i: (i, 0)),
        pl.BlockSpec((1, D), lambda i: (0, 0)),
    ],
    out_specs=pl.BlockSpec((BN, D), lambda i: (i, 0)),
    out_shape=jax.ShapeDtypeStruct((N, D), jnp.float32),
)


def kernel(x, edge_index, W1, b1, W2, b2):
    row = edge_index[0].astype(jnp.int32)
    col = edge_index[1].astype(jnp.int32)
    pad = E_PAD - E
    row_p = jnp.concatenate([row, jnp.zeros((pad,), jnp.int32)])
    col_p = jnp.concatenate([col, jnp.full((pad,), N, jnp.int32)])
    row_r = row_p.reshape(NS, CPT, CHUNK)
    col_r = col_p.reshape(NS, CPT, CHUNK)
    zeros1 = jnp.zeros((RPT,), jnp.float32)
    zeros2 = jnp.zeros((RPT, DH), jnp.float32)

    deg_parts = _deg_kernel(col_r, zeros1)
    xw1 = _tc1a(x, W1)
    deg1 = deg_parts[:N].reshape(N, 1)
    xw1s, dis = _tc1b(deg1, xw1)

    acc1 = _msg_kernel(xw1s, row_r, col_r, zeros2)
    xw2s = _tc2(acc1, acc1, xw1s, xw1s, dis, b1.reshape(1, D), W2)

    acc2 = _msg_kernel(xw2s, row_r, col_r, zeros2)
    out = _tc3(acc2, acc2, xw2s, xw2s, dis, b2.reshape(1, D))
    return out


# speedup vs baseline: 1.0233x; 1.0233x over previous
import functools

import jax
import jax.numpy as jnp
from jax import lax
from jax.experimental import pallas as pl
from jax.experimental.pallas import tpu as pltpu
from jax.experimental.pallas import tpu_sc as plsc

N = 10000
E = 320000
D = 128
DH = D // 2

NC = 2
NS = 16
CHUNK = 128
SUP = 4

_cpt = -(-E // (NS * CHUNK))
CPT = _cpt + (1 - _cpt % 2)
E_PAD = NS * CPT * CHUNK

RPT = -(-(N + 1) // (NS * 128)) * 128
NPAD = RPT * NS

_MESH = plsc.VectorSubcoreMesh(core_axis_name="c", subcore_axis_name="s")


@functools.partial(
    pl.kernel,
    out_type=jax.ShapeDtypeStruct((NC * NPAD,), jnp.float32),
    mesh=_MESH,
    scratch_types=[
        pltpu.VMEM((CPT, CHUNK), jnp.int32),
        pltpu.VMEM((CHUNK,), jnp.float32),
        pltpu.VMEM_SHARED((NPAD,), jnp.float32),
    ],
)
def _deg_kernel(col_hbm, zeros1_hbm, out_hbm, col_v, ones_v, acc):
    c = lax.axis_index("c")
    s = lax.axis_index("s")
    pltpu.sync_copy(col_hbm.at[s], col_v)
    for i in range(CHUNK // 16):
        ones_v[pl.ds(i * 16, 16)] = jnp.ones((16,), jnp.float32)
    pltpu.sync_copy(zeros1_hbm, acc.at[pl.ds(s * RPT, RPT)])
    plsc.subcore_barrier()

    def body(j, carry):
        pltpu.sync_copy(ones_v, acc.at[col_v.at[j]], add=True)
        return carry

    lax.fori_loop(0, CPT, body, 0)
    plsc.subcore_barrier()
    pltpu.sync_copy(acc.at[pl.ds(s * RPT, RPT)],
                    out_hbm.at[pl.ds(c * NPAD + s * RPT, RPT)])


@functools.partial(
    pl.kernel,
    out_type=jax.ShapeDtypeStruct((NC, NPAD, DH), jnp.float32),
    mesh=_MESH,
    scratch_types=[
        pltpu.VMEM((CPT, CHUNK), jnp.int32),
        pltpu.VMEM((CPT, CHUNK), jnp.int32),
        pltpu.VMEM((CHUNK, DH), jnp.float32),
        pltpu.VMEM((CHUNK, DH), jnp.float32),
        pltpu.VMEM_SHARED((NPAD, DH), jnp.float32),
        pltpu.SemaphoreType.DMA,
        pltpu.SemaphoreType.DMA,
    ],
    compiler_params=pltpu.CompilerParams(use_tc_tiling_on_sc=False),
)
def _msg_kernel(xw_hbm, row_hbm, col_hbm, zeros2_hbm, out_hbm,
                row_v, col_v, buf0, buf1, acc, sem0, sem1):
    c = lax.axis_index("c")
    s = lax.axis_index("s")
    xw_c = xw_hbm.at[c]
    pltpu.async_copy(row_hbm.at[s], row_v, sem0)
    pltpu.async_copy(col_hbm.at[s], col_v, sem1)
    pltpu.sync_copy(zeros2_hbm, acc.at[pl.ds(s * RPT, RPT)])
    pltpu.make_async_copy(row_hbm.at[s], row_v, sem0).wait()
    pltpu.make_async_copy(col_hbm.at[s], col_v, sem1).wait()
    plsc.subcore_barrier()

    pltpu.async_copy(xw_c.at[row_v.at[0]], buf0, sem0)

    def pair(i, carry):
        j = 2 * i
        pltpu.make_async_copy(xw_c.at[row_v.at[j]], buf0, sem0).wait()
        pltpu.async_copy(xw_c.at[row_v.at[j + 1]], buf1, sem1)
        pltpu.sync_copy(buf0, acc.at[col_v.at[j]], add=True)
        pltpu.make_async_copy(xw_c.at[row_v.at[j + 1]], buf1, sem1).wait()
        pltpu.async_copy(xw_c.at[row_v.at[j + 2]], buf0, sem0)
        pltpu.sync_copy(buf1, acc.at[col_v.at[j + 1]], add=True)
        return carry

    lax.fori_loop(0, (CPT - 1) // 2, pair, 0)
    pltpu.make_async_copy(xw_c.at[row_v.at[CPT - 1]], buf0, sem0).wait()
    pltpu.sync_copy(buf0, acc.at[col_v.at[CPT - 1]], add=True)

    plsc.subcore_barrier()
    pltpu.sync_copy(acc.at[pl.ds(s * RPT, RPT)],
                    out_hbm.at[c, pl.ds(s * RPT, RPT)])


BN = 2000


def _tc1a_body(x_ref, w1_ref, xw_ref):
    xw_ref[...] = jnp.dot(x_ref[...], w1_ref[...],
                          preferred_element_type=jnp.float32)


_tc1a = pl.pallas_call(
    _tc1a_body,
    grid=(N // BN,),
    in_specs=[
        pl.BlockSpec((BN, D), lambda i: (i, 0)),
        pl.BlockSpec((D, D), lambda i: (0, 0)),
    ],
    out_specs=pl.BlockSpec((BN, D), lambda i: (i, 0)),
    out_shape=jax.ShapeDtypeStruct((N, D), jnp.float32),
)


def _tc1b_body(deg_ref, xw_ref, xws_ref, dis_ref):
    dis = lax.rsqrt(deg_ref[...] + 1.0)
    v = dis * xw_ref[...]
    xws_ref[0] = v[:, :DH]
    xws_ref[1] = v[:, DH:]
    dis_ref[...] = dis


_tc1b = pl.pallas_call(
    _tc1b_body,
    grid=(N // BN,),
    in_specs=[
        pl.BlockSpec((BN, 1), lambda i: (i, 0)),
        pl.BlockSpec((BN, D), lambda i: (i, 0)),
    ],
    out_specs=[
        pl.BlockSpec((NC, BN, DH), lambda i: (0, i, 0)),
        pl.BlockSpec((BN, 1), lambda i: (i, 0)),
    ],
    out_shape=[
        jax.ShapeDtypeStruct((NC, N, DH), jnp.float32),
        jax.ShapeDtypeStruct((N, 1), jnp.float32),
    ],
)


def _tc2_body(a0_ref, a1_ref, x0_ref, x1_ref, dis_ref, b_ref, w_ref, out_ref):
    dis = dis_ref[...]
    pre = jnp.concatenate(
        [a0_ref[0] + x0_ref[0], a1_ref[0] + x1_ref[0]], axis=1)
    h = jnp.maximum(dis * pre + b_ref[...], 0.0)
    v = dis * jnp.dot(h, w_ref[...], preferred_element_type=jnp.float32)
    out_ref[0] = v[:, :DH]
    out_ref[1] = v[:, DH:]


_tc2 = pl.pallas_call(
    _tc2_body,
    grid=(N // BN,),
    in_specs=[
        pl.BlockSpec((1, BN, DH), lambda i: (0, i, 0)),
        pl.BlockSpec((1, BN, DH), lambda i: (1, i, 0)),
        pl.BlockSpec((1, BN, DH), lambda i: (0, i, 0)),
        pl.BlockSpec((1, BN, DH), lambda i: (1, i, 0)),
        pl.BlockSpec((BN, 1), lambda i: (i, 0)),
        pl.BlockSpec((1, D), lambda i: (0, 0)),
        pl.BlockSpec((D, D), lambda i: (0, 0)),
    ],
    out_specs=pl.BlockSpec((NC, BN, DH), lambda i: (0, i, 0)),
    out_shape=jax.ShapeDtypeStruct((NC, N, DH), jnp.float32),
)


def _tc3_body(a0_ref, a1_ref, x0_ref, x1_ref, dis_ref, b_ref, out_ref):
    dis = dis_ref[...]
    pre = jnp.concatenate(
        [a0_ref[0] + x0_ref[0], a1_ref[0] + x1_ref[0]], axis=1)
    out_ref[...] = jnp.maximum(dis * pre + b_ref[...], 0.0)


_tc3 = pl.pallas_call(
    _tc3_body,
    grid=(N // BN,),
    in_specs=[
        pl.BlockSpec((1, BN, DH), lambda i: (0, i, 0)),
        pl.BlockSpec((1, BN, DH), lambda i: (1, i, 0)),
        pl.BlockSpec((1, BN, DH), lambda i: (0, i, 0)),
        pl.BlockSpec((1, BN, DH), lambda i: (1, i, 0)),
        pl.BlockSpec((BN, 1), lambda i: (i, 0)),
        pl.BlockSpec((1, D), lambda i: (0, 0)),
    ],
    out_specs=pl.BlockSpec((BN, D), lambda i: (i, 0)),
    out_shape=jax.ShapeDtypeStruct((N, D), jnp.float32),
)


def kernel(x, edge_index, W1, b1, W2, b2):
    row = edge_index[0].astype(jnp.int32)
    col = edge_index[1].astype(jnp.int32)
    pad = E_PAD - E
    row_p = jnp.concatenate([row, jnp.zeros((pad,), jnp.int32)])
    col_p = jnp.concatenate([col, jnp.full((pad,), N, jnp.int32)])
    row_r = row_p.reshape(NS, CPT, CHUNK)
    col_r = col_p.reshape(NS, CPT, CHUNK)
    zeros1 = jnp.zeros((RPT,), jnp.float32)
    zeros2 = jnp.zeros((RPT, DH), jnp.float32)

    deg_parts = _deg_kernel(col_r, zeros1)
    xw1 = _tc1a(x, W1)
    deg1 = deg_parts[:N].reshape(N, 1)
    xw1s, dis = _tc1b(deg1, xw1)

    acc1 = _msg_kernel(xw1s, row_r, col_r, zeros2)
    xw2s = _tc2(acc1, acc1, xw1s, xw1s, dis, b1.reshape(1, D), W2)

    acc2 = _msg_kernel(xw2s, row_r, col_r, zeros2)
    out = _tc3(acc2, acc2, xw2s, xw2s, dis, b2.reshape(1, D))
    return out
